# baseline (speedup 1.0000x reference)
import functools

import numpy as np

import jax
import jax.numpy as jnp
from jax import lax
from jax.experimental import pallas as pl
from jax.experimental.pallas import tpu as pltpu
from jax.experimental.pallas import tpu_sc as plsc

N_NODES = 10240
N_REAL = 10000
D = 128
NC = 2
NS = 16
NW = NC * NS
CHUNK = 128
QB = 8
NBUF = 2
ZROWS = 64


def _sc_segment_sum(n_blocks):
    rows_per_tile = N_NODES // NS

    mesh = plsc.VectorSubcoreMesh(
        core_axis_name="c", subcore_axis_name="s", num_cores=NC, num_subcores=NS
    )

    @functools.partial(
        pl.kernel,
        out_type=jax.ShapeDtypeStruct((NC, N_NODES, D), jnp.float32),
        mesh=mesh,
        scratch_types=[
            [[pltpu.VMEM((QB, CHUNK), jnp.int32) for _ in range(2)]
             for _ in range(2)],
            [pltpu.VMEM((CHUNK, D), jnp.float32) for _ in range(NBUF)],
            pltpu.VMEM((ZROWS, D), jnp.float32),
            [pltpu.SemaphoreType.DMA for _ in range(2)],
            [pltpu.SemaphoreType.DMA for _ in range(NBUF)],
            pltpu.VMEM_SHARED((N_NODES, D), jnp.float32),
        ],
    )
    def kern(x_hbm, src_hbm, dst_hbm, out_hbm,
             idxs, rows, zbuf, isems, gsems, agg):
        cid = lax.axis_index("c")
        sid = lax.axis_index("s")
        wid = cid * NS + sid

        zvec = jnp.zeros((16,), jnp.float32)

        def zrow(i, _):
            for k in range(D // 16):
                zbuf[i, pl.ds(k * 16, 16)] = zvec
            return 0

        lax.fori_loop(0, ZROWS, zrow, 0)
        r0 = sid * rows_per_tile
        nz = rows_per_tile // ZROWS
        for c in range(nz):
            pltpu.async_copy(zbuf, agg.at[pl.ds(r0 + c * ZROWS, ZROWS)],
                             isems[0])
        for c in range(nz):
            pltpu.make_async_copy(zbuf, agg.at[pl.ds(r0 + c * ZROWS, ZROWS)],
                                  isems[0]).wait()
        plsc.subcore_barrier()

        def load_idx(t, iset, sem):
            pltpu.async_copy(src_hbm.at[wid, t], iset[0], sem)
            pltpu.async_copy(dst_hbm.at[wid, t], iset[1], sem)

        def wait_idx(iset, sem):
            pltpu.make_async_copy(src_hbm.at[wid, 0], iset[0], sem).wait()
            pltpu.make_async_copy(dst_hbm.at[wid, 0], iset[1], sem).wait()

        load_idx(0, idxs[0], isems[0])
        wait_idx(idxs[0], isems[0])
        load_idx(1, idxs[1], isems[1])
        for b in range(NBUF):
            pltpu.async_copy(x_hbm.at[idxs[0][0].at[b]], rows[b], gsems[b])

        def outer(u, _):
            for half in range(2):
                t = 2 * u + half
                iset, inext = idxs[half], idxs[1 - half]

                for q in range(QB):
                    b = q % NBUF
                    pltpu.make_async_copy(
                        x_hbm.at[iset[0].at[q]], rows[b], gsems[b]).wait()

                    if q == QB - NBUF:
                        @pl.when(t + 1 < n_blocks)
                        def _():
                            wait_idx(inext, isems[1 - half])

                    pltpu.sync_copy(rows[b], agg.at[iset[1].at[q]], add=True)

                    r = q + NBUF
                    if r < QB:
                        pltpu.async_copy(
                            x_hbm.at[iset[0].at[r]], rows[b], gsems[b])
                    else:
                        @pl.when(t + 1 < n_blocks)
                        def _():
                            pltpu.async_copy(
                                x_hbm.at[inext[0].at[r - QB]], rows[b],
                                gsems[b])

                @pl.when(t + 2 < n_blocks)
                def _():
                    load_idx(t + 2, iset, isems[half])
            return 0

        lax.fori_loop(0, n_blocks // 2, outer, 0)
        plsc.subcore_barrier()

        pltpu.sync_copy(agg.at[pl.ds(r0, rows_per_tile)],
                        out_hbm.at[cid, pl.ds(r0, rows_per_tile)])

    return kern


def _combine_body(p_ref, w_ref, b_ref, o_ref):
    s = p_ref[0] + p_ref[1]
    o_ref[...] = (
        jnp.dot(s, w_ref[...], preferred_element_type=jnp.float32) + b_ref[...]
    )


def kernel(x, edge_index, W, b):
    n = x.shape[0]
    e = edge_index.shape[1]
    d = x.shape[1]

    src = edge_index[0].astype(jnp.int32)
    dst = edge_index[1].astype(jnp.int32)

    group = NW * 2 * QB * CHUNK
    e_pad = -(-e // group) * group
    n_chunks = e_pad // (NW * CHUNK)
    n_blocks = n_chunks // QB
    pad = e_pad - e
    pad_idx = np.arange(pad, dtype=np.int32)
    pad_src = jnp.asarray(pad_idx % min(n, 4096))
    pad_dst = jnp.asarray(N_REAL + pad_idx % (N_NODES - N_REAL))
    src = jnp.concatenate([src, pad_src]).reshape(NW, n_blocks, QB, CHUNK)
    dst = jnp.concatenate([dst, pad_dst]).reshape(NW, n_blocks, QB, CHUNK)

    partials = _sc_segment_sum(n_blocks)(x, src, dst)

    bm = 2048
    out = pl.pallas_call(
        _combine_body,
        grid=(-(-n // bm),),
        in_specs=[
            pl.BlockSpec((NC, bm, d), lambda i: (0, i, 0)),
            pl.BlockSpec((d, W.shape[1]), lambda i: (0, 0)),
            pl.BlockSpec((1, W.shape[1]), lambda i: (0, 0)),
        ],
        out_specs=pl.BlockSpec((bm, W.shape[1]), lambda i: (i, 0)),
        out_shape=jax.ShapeDtypeStruct((n, W.shape[1]), jnp.float32),
    )(partials, W, b)
    return out


# speedup vs baseline: 1.0670x; 1.0670x over previous
import functools

import numpy as np

import jax
import jax.numpy as jnp
from jax import lax
from jax.experimental import pallas as pl
from jax.experimental.pallas import tpu as pltpu
from jax.experimental.pallas import tpu_sc as plsc

N_NODES = 10240
N_REAL = 10000
D = 128
NC = 2
NS = 16
NW = NC * NS
CHUNK = 128
QB = 8
NBUF = 2
ZROWS = 64


def _sc_segment_sum(n_blocks):
    rows_per_tile = N_NODES // NS

    mesh = plsc.VectorSubcoreMesh(
        core_axis_name="c", subcore_axis_name="s", num_cores=NC, num_subcores=NS
    )

    @functools.partial(
        pl.kernel,
        out_type=jax.ShapeDtypeStruct((NC, N_NODES, D), jnp.float32),
        mesh=mesh,
        scratch_types=[
            [[pltpu.VMEM((QB, CHUNK), jnp.int32) for _ in range(2)]
             for _ in range(2)],
            [pltpu.VMEM((CHUNK, D), jnp.float32) for _ in range(NBUF)],
            pltpu.VMEM((ZROWS, D), jnp.float32),
            [pltpu.SemaphoreType.DMA for _ in range(2)],
            [pltpu.SemaphoreType.DMA for _ in range(NBUF)],
            pltpu.VMEM_SHARED((N_NODES, D), jnp.float32),
        ],
    )
    def kern(x_hbm, sd_hbm, out_hbm, idxs, rows, zbuf, isems, gsems, agg):
        cid = lax.axis_index("c")
        sid = lax.axis_index("s")
        wid = cid * NS + sid

        zvec = jnp.zeros((16,), jnp.float32)

        def zrow(i, _):
            for k in range(D // 16):
                zbuf[i, pl.ds(k * 16, 16)] = zvec
            return 0

        lax.fori_loop(0, ZROWS, zrow, 0)
        r0 = sid * rows_per_tile
        nz = rows_per_tile // ZROWS
        for c in range(nz):
            pltpu.async_copy(zbuf, agg.at[pl.ds(r0 + c * ZROWS, ZROWS)],
                             isems[0])
        for c in range(nz):
            pltpu.make_async_copy(zbuf, agg.at[pl.ds(r0 + c * ZROWS, ZROWS)],
                                  isems[0]).wait()
        plsc.subcore_barrier()

        def load_idx(t, iset, sem):
            pltpu.async_copy(sd_hbm.at[0, wid, t], iset[0], sem)
            pltpu.async_copy(sd_hbm.at[1, wid, t], iset[1], sem)

        def wait_idx(iset, sem):
            pltpu.make_async_copy(sd_hbm.at[0, wid, 0], iset[0], sem).wait()
            pltpu.make_async_copy(sd_hbm.at[1, wid, 0], iset[1], sem).wait()

        load_idx(0, idxs[0], isems[0])
        wait_idx(idxs[0], isems[0])
        load_idx(1, idxs[1], isems[1])
        for b in range(NBUF):
            pltpu.async_copy(x_hbm.at[idxs[0][0].at[b]], rows[b], gsems[b])

        def outer(u, _):
            for half in range(2):
                t = 2 * u + half
                iset, inext = idxs[half], idxs[1 - half]

                for q in range(QB):
                    b = q % NBUF
                    pltpu.make_async_copy(
                        x_hbm.at[iset[0].at[q]], rows[b], gsems[b]).wait()

                    if q == QB - NBUF:
                        @pl.when(t + 1 < n_blocks)
                        def _():
                            wait_idx(inext, isems[1 - half])

                    pltpu.sync_copy(rows[b], agg.at[iset[1].at[q]], add=True)

                    r = q + NBUF
                    if r < QB:
                        pltpu.async_copy(
                            x_hbm.at[iset[0].at[r]], rows[b], gsems[b])
                    else:
                        @pl.when(t + 1 < n_blocks)
                        def _():
                            pltpu.async_copy(
                                x_hbm.at[inext[0].at[r - QB]], rows[b],
                                gsems[b])

                @pl.when(t + 2 < n_blocks)
                def _():
                    load_idx(t + 2, iset, isems[half])
            return 0

        lax.fori_loop(0, n_blocks // 2, outer, 0)
        plsc.subcore_barrier()

        pltpu.sync_copy(agg.at[pl.ds(r0, rows_per_tile)],
                        out_hbm.at[cid, pl.ds(r0, rows_per_tile)])

    return kern


def _combine_body(p_ref, w_ref, b_ref, o_ref):
    s = p_ref[0] + p_ref[1]
    o_ref[...] = (
        jnp.dot(s, w_ref[...], preferred_element_type=jnp.float32) + b_ref[...]
    )


def kernel(x, edge_index, W, b):
    n = x.shape[0]
    e = edge_index.shape[1]
    d = x.shape[1]

    ei = edge_index.astype(jnp.int32)

    group = NW * 2 * QB * CHUNK
    e_pad = -(-e // group) * group
    n_chunks = e_pad // (NW * CHUNK)
    n_blocks = n_chunks // QB
    pad = e_pad - e
    pad_idx = np.arange(pad, dtype=np.int32)
    pad_block = np.stack([pad_idx % min(n, 4096),
                          N_REAL + pad_idx % (N_NODES - N_REAL)])
    sd = jnp.concatenate([ei, jnp.asarray(pad_block)], axis=1).reshape(
        2, NW, n_blocks, QB, CHUNK)

    partials = _sc_segment_sum(n_blocks)(x, sd)

    bm = 2048
    out = pl.pallas_call(
        _combine_body,
        grid=(-(-n // bm),),
        in_specs=[
            pl.BlockSpec((NC, bm, d), lambda i: (0, i, 0)),
            pl.BlockSpec((d, W.shape[1]), lambda i: (0, 0)),
            pl.BlockSpec((1, W.shape[1]), lambda i: (0, 0)),
        ],
        out_specs=pl.BlockSpec((bm, W.shape[1]), lambda i: (i, 0)),
        out_shape=jax.ShapeDtypeStruct((n, W.shape[1]), jnp.float32),
    )(partials, W, b)
    return out


# speedup vs baseline: 1.0895x; 1.0211x over previous
import functools

import numpy as np

import jax
import jax.numpy as jnp
from jax import lax
from jax.experimental import pallas as pl
from jax.experimental.pallas import tpu as pltpu
from jax.experimental.pallas import tpu_sc as plsc

N_NODES = 10240
N_REAL = 10000
D = 128
NC = 2
NS = 16
NW = NC * NS
CHUNK = 128
QB = 8
NBUF = 2
ZROWS = 64


def _sc_segment_sum(n_blocks):
    rows_per_tile = N_NODES // NS

    mesh = plsc.VectorSubcoreMesh(
        core_axis_name="c", subcore_axis_name="s", num_cores=NC, num_subcores=NS
    )

    @functools.partial(
        pl.kernel,
        out_type=jax.ShapeDtypeStruct((NC, N_NODES, D), jnp.float32),
        mesh=mesh,
        scratch_types=[
            [[pltpu.VMEM((QB, CHUNK), jnp.int32) for _ in range(2)]
             for _ in range(2)],
            [pltpu.VMEM((CHUNK, D), jnp.float32) for _ in range(NBUF)],
            pltpu.VMEM((ZROWS, D), jnp.float32),
            [pltpu.SemaphoreType.DMA for _ in range(2)],
            [pltpu.SemaphoreType.DMA for _ in range(NBUF)],
            pltpu.SemaphoreType.DMA,
            pltpu.VMEM_SHARED((N_NODES, D), jnp.float32),
        ],
    )
    def kern(x_hbm, sd_hbm, out_hbm, idxs, rows, zbuf, isems, gsems, zsem,
             agg):
        cid = lax.axis_index("c")
        sid = lax.axis_index("s")
        wid = cid * NS + sid

        zvec = jnp.zeros((16,), jnp.float32)

        def zrow(i, _):
            for k in range(D // 16):
                zbuf[i, pl.ds(k * 16, 16)] = zvec
            return 0

        lax.fori_loop(0, ZROWS, zrow, 0)
        r0 = sid * rows_per_tile
        nz = rows_per_tile // ZROWS
        for c in range(nz):
            pltpu.async_copy(zbuf, agg.at[pl.ds(r0 + c * ZROWS, ZROWS)],
                             zsem)

        def load_idx(t, iset, sem):
            pltpu.async_copy(sd_hbm.at[0, wid, t], iset[0], sem)
            pltpu.async_copy(sd_hbm.at[1, wid, t], iset[1], sem)

        def wait_idx(iset, sem):
            pltpu.make_async_copy(sd_hbm.at[0, wid, 0], iset[0], sem).wait()
            pltpu.make_async_copy(sd_hbm.at[1, wid, 0], iset[1], sem).wait()

        load_idx(0, idxs[0], isems[0])
        wait_idx(idxs[0], isems[0])
        load_idx(1, idxs[1], isems[1])
        for b in range(NBUF):
            pltpu.async_copy(x_hbm.at[idxs[0][0].at[b]], rows[b], gsems[b])

        for c in range(nz):
            pltpu.make_async_copy(zbuf, agg.at[pl.ds(r0 + c * ZROWS, ZROWS)],
                                  zsem).wait()
        plsc.subcore_barrier()

        def outer(u, _):
            for half in range(2):
                t = 2 * u + half
                iset, inext = idxs[half], idxs[1 - half]

                for q in range(QB):
                    b = q % NBUF
                    pltpu.make_async_copy(
                        x_hbm.at[iset[0].at[q]], rows[b], gsems[b]).wait()

                    if q == QB - NBUF:
                        @pl.when(t + 1 < n_blocks)
                        def _():
                            wait_idx(inext, isems[1 - half])

                    pltpu.sync_copy(rows[b], agg.at[iset[1].at[q]], add=True)

                    r = q + NBUF
                    if r < QB:
                        pltpu.async_copy(
                            x_hbm.at[iset[0].at[r]], rows[b], gsems[b])
                    else:
                        @pl.when(t + 1 < n_blocks)
                        def _():
                            pltpu.async_copy(
                                x_hbm.at[inext[0].at[r - QB]], rows[b],
                                gsems[b])

                @pl.when(t + 2 < n_blocks)
                def _():
                    load_idx(t + 2, iset, isems[half])
            return 0

        lax.fori_loop(0, n_blocks // 2, outer, 0)
        plsc.subcore_barrier()

        pltpu.sync_copy(agg.at[pl.ds(r0, rows_per_tile)],
                        out_hbm.at[cid, pl.ds(r0, rows_per_tile)])

    return kern


def _combine_body(p_ref, w_ref, b_ref, o_ref):
    s = p_ref[0] + p_ref[1]
    o_ref[...] = (
        jnp.dot(s, w_ref[...], preferred_element_type=jnp.float32) + b_ref[...]
    )


def kernel(x, edge_index, W, b):
    n = x.shape[0]
    e = edge_index.shape[1]
    d = x.shape[1]

    ei = edge_index.astype(jnp.int32)

    group = NW * 2 * QB * CHUNK
    e_pad = -(-e // group) * group
    n_chunks = e_pad // (NW * CHUNK)
    n_blocks = n_chunks // QB
    pad = e_pad - e
    pad_idx = np.arange(pad, dtype=np.int32)
    pad_block = np.stack([pad_idx % min(n, 4096),
                          N_REAL + pad_idx % (N_NODES - N_REAL)])
    sd = jnp.concatenate([ei, jnp.asarray(pad_block)], axis=1).reshape(
        2, NW, n_blocks, QB, CHUNK)

    partials = _sc_segment_sum(n_blocks)(x, sd)

    bm = 2048
    out = pl.pallas_call(
        _combine_body,
        grid=(-(-n // bm),),
        in_specs=[
            pl.BlockSpec((NC, bm, d), lambda i: (0, i, 0)),
            pl.BlockSpec((d, W.shape[1]), lambda i: (0, 0)),
            pl.BlockSpec((1, W.shape[1]), lambda i: (0, 0)),
        ],
        out_specs=pl.BlockSpec((bm, W.shape[1]), lambda i: (i, 0)),
        out_shape=jax.ShapeDtypeStruct((n, W.shape[1]), jnp.float32),
    )(partials, W, b)
    return out
